# baseline (speedup 1.0000x reference)
import jax
import jax.numpy as jnp
from jax.experimental import pallas as pl
from jax.experimental.pallas import tpu as pltpu

_ROWS, _COLS = 16384, 128
_CHUNK = 1024
_NCHUNK = _ROWS // _CHUNK


def _bce_mean_kernel(x_hbm, z_hbm, out_ref, xb, zb, sem):
    def x_copy(k):
        return pltpu.make_async_copy(
            x_hbm.at[pl.ds(k * _CHUNK, _CHUNK), :], xb.at[k], sem.at[k, 0])

    def z_copy(k):
        return pltpu.make_async_copy(
            z_hbm.at[pl.ds(k * _CHUNK, _CHUNK), :], zb.at[k], sem.at[k, 1])

    for k in range(_NCHUNK):
        x_copy(k).start()
        z_copy(k).start()

    ones = jnp.ones((8, _CHUNK), jnp.float32)
    acc = jnp.zeros((8, _COLS), jnp.float32)
    for k in range(_NCHUNK):
        x_copy(k).wait()
        z_copy(k).wait()
        x = xb[k]
        z = zb[k]
        bce = jnp.maximum(x, 0.0) - x * z + jnp.log(1.0 + jnp.exp(-jnp.abs(x)))
        acc = acc + jax.lax.dot_general(
            ones, bce, (((1,), (0,)), ((), ())),
            preferred_element_type=jnp.float32,
        )

    out_ref[0, 0] = jnp.sum(acc) * (1.0 / (8 * _ROWS * _COLS))


def kernel(pred, target):
    out = pl.pallas_call(
        _bce_mean_kernel,
        in_specs=[
            pl.BlockSpec(memory_space=pltpu.MemorySpace.HBM),
            pl.BlockSpec(memory_space=pltpu.MemorySpace.HBM),
        ],
        out_specs=pl.BlockSpec(memory_space=pltpu.SMEM),
        out_shape=jax.ShapeDtypeStruct((1, 1), jnp.float32),
        scratch_shapes=[
            pltpu.VMEM((_NCHUNK, _CHUNK, _COLS), jnp.float32),
            pltpu.VMEM((_NCHUNK, _CHUNK, _COLS), jnp.float32),
            pltpu.SemaphoreType.DMA((_NCHUNK, 2)),
        ],
    )(pred, target)
    return out[0, 0]


# speedup vs baseline: 1.0424x; 1.0424x over previous
import jax
import jax.numpy as jnp
from jax.experimental import pallas as pl
from jax.experimental.pallas import tpu as pltpu

_ROWS, _COLS = 16384, 128
_CHUNK = 2048
_NCHUNK = _ROWS // _CHUNK


def _bce_mean_kernel(x_hbm, z_hbm, out_ref, xb, zb, sem):
    def x_copy(k):
        return pltpu.make_async_copy(
            x_hbm.at[pl.ds(k * _CHUNK, _CHUNK), :], xb.at[k], sem.at[k, 0])

    def z_copy(k):
        return pltpu.make_async_copy(
            z_hbm.at[pl.ds(k * _CHUNK, _CHUNK), :], zb.at[k], sem.at[k, 1])

    for k in range(_NCHUNK):
        x_copy(k).start()
        z_copy(k).start()

    ones = jnp.ones((8, _CHUNK), jnp.float32)
    acc = jnp.zeros((8, _COLS), jnp.float32)
    for k in range(_NCHUNK):
        x_copy(k).wait()
        z_copy(k).wait()
        x = xb[k]
        z = zb[k]
        bce = jnp.maximum(x, 0.0) - x * z + jnp.log(1.0 + jnp.exp(-jnp.abs(x)))
        acc = acc + jax.lax.dot_general(
            ones, bce, (((1,), (0,)), ((), ())),
            preferred_element_type=jnp.float32,
        )

    out_ref[0, 0] = jnp.sum(acc) * (1.0 / (8 * _ROWS * _COLS))


def kernel(pred, target):
    out = pl.pallas_call(
        _bce_mean_kernel,
        in_specs=[
            pl.BlockSpec(memory_space=pltpu.MemorySpace.HBM),
            pl.BlockSpec(memory_space=pltpu.MemorySpace.HBM),
        ],
        out_specs=pl.BlockSpec(memory_space=pltpu.SMEM),
        out_shape=jax.ShapeDtypeStruct((1, 1), jnp.float32),
        scratch_shapes=[
            pltpu.VMEM((_NCHUNK, _CHUNK, _COLS), jnp.float32),
            pltpu.VMEM((_NCHUNK, _CHUNK, _COLS), jnp.float32),
            pltpu.SemaphoreType.DMA((_NCHUNK, 2)),
        ],
    )(pred, target)
    return out[0, 0]
